# Initial kernel scaffold; baseline (speedup 1.0000x reference)
import jax
import jax.numpy as jnp
from jax.experimental import pallas as pl


def kernel(x_ble, W_router, b_router, gate_nfe, enc_nfe, dec_nef):
    raise NotImplementedError("write your pallas kernel here")


# speedup vs baseline: 1.7619x; 1.7619x over previous
import functools

import jax
import jax.numpy as jnp
from jax import lax
from jax.experimental import pallas as pl
from jax.experimental.pallas import tpu as pltpu
from jax.experimental.pallas import tpu_sc as plsc

EMBED = 1024
FF = 1024
NEXP = 64
CAP = 128
TOKENS = 8192
CHUNK = 512
NCHUNK = TOKENS // CHUNK
OVERFLOW = NEXP * CAP
ROWS_PAD = OVERFLOW + CAP

NC, NS = 2, 16
NW = NC * NS
TPW = TOKENS // NW
JCH = 4
JW = TPW // JCH


def _router_body(x_ref, w_ref, b_ref, slots_ref, loss_ref, counts_ref, imp_ref):
    i = pl.program_id(0)

    @pl.when(i == 0)
    def _init():
        counts_ref[...] = jnp.zeros_like(counts_ref)
        imp_ref[...] = jnp.zeros_like(imp_ref)

    x = x_ref[...]
    w = w_ref[...]
    logits = lax.dot_general(x, w, (((1,), (1,)), ((), ())),
                             preferred_element_type=jnp.float32)
    logits = logits + b_ref[...]

    m = jnp.max(logits, axis=1, keepdims=True)
    p = jnp.exp(logits - m)
    scores = p / jnp.sum(p, axis=1, keepdims=True)
    imp_ref[...] += jnp.sum(scores, axis=0, keepdims=True)

    col = lax.broadcasted_iota(jnp.int32, (CHUNK, NEXP), 1)
    amax = jnp.min(jnp.where(logits == m, col, NEXP), axis=1)
    oh = (col == amax[:, None]).astype(jnp.float32)

    r_i = lax.broadcasted_iota(jnp.int32, (CHUNK, CHUNK), 0)
    c_i = lax.broadcasted_iota(jnp.int32, (CHUNK, CHUNK), 1)
    ltri = (r_i > c_i).astype(jnp.float32)
    within = lax.dot_general(ltri, oh, (((1,), (0,)), ((), ())),
                             preferred_element_type=jnp.float32)
    rank_f = (jnp.sum(within * oh, axis=1)
              + jnp.sum(oh * counts_ref[0, :][None, :], axis=1))
    counts_ref[...] += jnp.sum(oh, axis=0, keepdims=True)

    rank = rank_f.astype(jnp.int32)
    slot = jnp.where(rank < CAP, amax * CAP + rank, OVERFLOW)
    slots_ref[0, 0, :] = slot

    @pl.when(i == NCHUNK - 1)
    def _fin():
        counts = counts_ref[0, :]
        imp = imp_ref[0, :]
        loss = (NEXP / (float(TOKENS) * float(TOKENS))
                * jnp.sum(counts * imp))
        loss_ref[...] = jnp.broadcast_to(loss, loss_ref.shape)


_router_call = pl.pallas_call(
    _router_body,
    grid=(NCHUNK,),
    in_specs=[
        pl.BlockSpec((CHUNK, EMBED), lambda i: (i, 0)),
        pl.BlockSpec((NEXP, EMBED), lambda i: (0, 0)),
        pl.BlockSpec((1, NEXP), lambda i: (0, 0)),
    ],
    out_specs=[
        pl.BlockSpec((1, 1, CHUNK), lambda i: (i, 0, 0)),
        pl.BlockSpec((1, 128), lambda i: (0, 0)),
    ],
    out_shape=[
        jax.ShapeDtypeStruct((NCHUNK, 1, CHUNK), jnp.int32),
        jax.ShapeDtypeStruct((1, 128), jnp.float32),
    ],
    scratch_shapes=[
        pltpu.VMEM((1, NEXP), jnp.float32),
        pltpu.VMEM((1, NEXP), jnp.float32),
    ],
)


def _ffn_body(x_ref, g_ref, e_ref, d_ref, out_ref):
    i = pl.program_id(0)

    @pl.when(i < NEXP)
    def _compute():
        x = x_ref[...]
        g = g_ref[0]
        en = e_ref[0]
        de = d_ref[0]
        h1 = lax.dot_general(x, g, (((1,), (1,)), ((), ())),
                             preferred_element_type=jnp.float32)
        h2 = lax.dot_general(x, en, (((1,), (1,)), ((), ())),
                             preferred_element_type=jnp.float32)
        act = h1 * jax.nn.sigmoid(h1) * h2
        out_ref[...] = lax.dot_general(act, de, (((1,), (1,)), ((), ())),
                                       preferred_element_type=jnp.float32)

    @pl.when(i == NEXP)
    def _zero():
        out_ref[...] = jnp.zeros_like(out_ref)


_ffn_call = pl.pallas_call(
    _ffn_body,
    grid=(NEXP + 1,),
    in_specs=[
        pl.BlockSpec((CAP, EMBED), lambda i: (i, 0)),
        pl.BlockSpec((1, FF, EMBED), lambda i: (jnp.minimum(i, NEXP - 1), 0, 0)),
        pl.BlockSpec((1, FF, EMBED), lambda i: (jnp.minimum(i, NEXP - 1), 0, 0)),
        pl.BlockSpec((1, EMBED, FF), lambda i: (jnp.minimum(i, NEXP - 1), 0, 0)),
    ],
    out_specs=pl.BlockSpec((CAP, EMBED), lambda i: (i, 0)),
    out_shape=jax.ShapeDtypeStruct((ROWS_PAD, EMBED), jnp.float32),
)


@functools.cache
def _sc_kernels():
    mesh = plsc.VectorSubcoreMesh(core_axis_name="c", subcore_axis_name="s",
                                  num_cores=NC, num_subcores=NS)
    scratch = [
        pltpu.VMEM((JCH, JW), jnp.int32),
        pltpu.VMEM((JW, EMBED), jnp.float32),
        pltpu.SemaphoreType.DMA,
    ]

    @functools.partial(
        pl.kernel,
        out_type=jax.ShapeDtypeStruct((ROWS_PAD, EMBED), jnp.float32),
        mesh=mesh, scratch_types=scratch,
    )
    def dispatch_sc(x_hbm, slots_hbm, xdisp_hbm, idx_v, rows_v, sem):
        wid = lax.axis_index("s") * NC + lax.axis_index("c")
        pltpu.sync_copy(slots_hbm.at[wid], idx_v)
        for j in range(JCH):
            pltpu.sync_copy(x_hbm.at[pl.ds(wid * TPW + j * JW, JW)], rows_v)
            pltpu.async_copy(rows_v, xdisp_hbm.at[idx_v.at[j]], sem).wait()

    @functools.partial(
        pl.kernel,
        out_type=jax.ShapeDtypeStruct((TOKENS, EMBED), jnp.float32),
        mesh=mesh, scratch_types=scratch,
    )
    def combine_sc(out2_hbm, slots_hbm, out_hbm, idx_v, rows_v, sem):
        wid = lax.axis_index("s") * NC + lax.axis_index("c")
        pltpu.sync_copy(slots_hbm.at[wid], idx_v)
        for j in range(JCH):
            pltpu.async_copy(out2_hbm.at[idx_v.at[j]], rows_v, sem).wait()
            pltpu.sync_copy(rows_v, out_hbm.at[pl.ds(wid * TPW + j * JW, JW)])

    return dispatch_sc, combine_sc


def kernel(x_ble, W_router, b_router, gate_nfe, enc_nfe, dec_nef):
    b, l, e = x_ble.shape
    x_te = x_ble.reshape(b * l, e)
    slots16, loss = _router_call(x_te, W_router, b_router.reshape(1, NEXP))
    slots3 = slots16.reshape(NW, JCH, JW)
    dispatch_sc, combine_sc = _sc_kernels()
    x_disp = dispatch_sc(x_te, slots3)
    out2 = _ffn_call(x_disp, gate_nfe, enc_nfe, dec_nef)
    out_te = combine_sc(out2, slots3)
    return out_te.reshape(b, l, e), loss[0, 0]
